# baseline (speedup 1.0000x reference)
import functools

import jax
import jax.numpy as jnp
from jax import lax
from jax.experimental import pallas as pl
from jax.experimental.pallas import tpu as pltpu
from jax.experimental.pallas import tpu_sc as plsc

_F = 26
_V1 = 100001
_D = 16
_B = 16384
_NC, _NS, _L = 2, 16, 16
_NW = _NC * _NS
_R = _B * _F
_RPW = _R // _NW
_CH = 1664
_NCHUNK = _RPW // _CH


def _body(tab_hbm, x_hbm, off_hbm, out_hbm, idx_v, off_v, buf0, buf1, sem0, sem1):
  wid = lax.axis_index("s") * _NC + lax.axis_index("c")
  base = wid * _RPW
  pltpu.sync_copy(x_hbm.at[pl.ds(base, _RPW)], idx_v)
  pltpu.sync_copy(off_hbm, off_v)

  def add_off(i, carry):
    s = pl.ds(i * _L, _L)
    idx_v[s] = idx_v[s] + off_v[s]
    return carry

  lax.fori_loop(0, _RPW // _L, add_off, 0, unroll=8)

  bufs = (buf0, buf1)
  sems = (sem0, sem1)
  pltpu.async_copy(tab_hbm.at[idx_v.at[pl.ds(0, _CH)]], bufs[0], sems[0])
  for c in range(_NCHUNK):
    if c + 1 < _NCHUNK:
      nxt = (c + 1) % 2
      pltpu.async_copy(
          tab_hbm.at[idx_v.at[pl.ds((c + 1) * _CH, _CH)]], bufs[nxt], sems[nxt]
      )
    cur = c % 2
    pltpu.make_async_copy(
        tab_hbm.at[idx_v.at[pl.ds(c * _CH, _CH)]], bufs[cur], sems[cur]
    ).wait()
    pltpu.sync_copy(bufs[cur], out_hbm.at[pl.ds(base + c * _CH, _CH)])


def kernel(X, tables):
  flat_tab = tables.reshape(_F * _V1, _D)
  x_flat = X.reshape(_R).astype(jnp.int32)
  off = jnp.tile(jnp.arange(_F, dtype=jnp.int32) * _V1, _RPW // _F)
  mesh = plsc.VectorSubcoreMesh(core_axis_name="c", subcore_axis_name="s")
  run = functools.partial(
      pl.kernel,
      out_type=jax.ShapeDtypeStruct((_R, _D), jnp.float32),
      mesh=mesh,
      compiler_params=pltpu.CompilerParams(use_tc_tiling_on_sc=False),
      scratch_types=[
          pltpu.VMEM((_RPW,), jnp.int32),
          pltpu.VMEM((_RPW,), jnp.int32),
          pltpu.VMEM((_CH, _D), jnp.float32),
          pltpu.VMEM((_CH, _D), jnp.float32),
          pltpu.SemaphoreType.DMA,
          pltpu.SemaphoreType.DMA,
      ],
  )(_body)
  out = run(flat_tab, x_flat, off)
  return out.reshape(_B, _F, _D)


# speedup vs baseline: 2.0326x; 2.0326x over previous
import functools

import jax
import jax.numpy as jnp
from jax import lax
from jax.experimental import pallas as pl
from jax.experimental.pallas import tpu as pltpu
from jax.experimental.pallas import tpu_sc as plsc

_F = 26
_V1 = 100001
_D = 16
_B = 16384
_NC, _NS, _L = 2, 16, 16
_NW = _NC * _NS
_BPW = _B // _NW


def _body(tab_hbm, x_hbm, out_hbm, idx0, idx1, buf0, buf1, sem0, sem1):
  wid = lax.axis_index("s") * _NC + lax.axis_index("c")
  b0 = wid * _BPW
  idxs = (idx0, idx1)
  bufs = (buf0, buf1)
  sems = (sem0, sem1)

  def start(f, k):
    pltpu.sync_copy(x_hbm.at[f, pl.ds(b0, _BPW)], idxs[k])
    pltpu.async_copy(tab_hbm.at[f].at[idxs[k]], bufs[k], sems[k])

  start(0, 0)
  for f in range(_F):
    if f + 1 < _F:
      start(f + 1, (f + 1) % 2)
    cur = f % 2
    pltpu.make_async_copy(
        tab_hbm.at[f].at[idxs[cur]], bufs[cur], sems[cur]
    ).wait()
    pltpu.sync_copy(bufs[cur], out_hbm.at[f, pl.ds(b0, _BPW)])


def kernel(X, tables):
  xt = jnp.transpose(X, (1, 0))
  mesh = plsc.VectorSubcoreMesh(core_axis_name="c", subcore_axis_name="s")
  run = functools.partial(
      pl.kernel,
      out_type=jax.ShapeDtypeStruct((_F, _B, _D), jnp.float32),
      mesh=mesh,
      compiler_params=pltpu.CompilerParams(use_tc_tiling_on_sc=False),
      scratch_types=[
          pltpu.VMEM((_BPW,), jnp.int32),
          pltpu.VMEM((_BPW,), jnp.int32),
          pltpu.VMEM((_BPW, _D), jnp.float32),
          pltpu.VMEM((_BPW, _D), jnp.float32),
          pltpu.SemaphoreType.DMA,
          pltpu.SemaphoreType.DMA,
      ],
  )(_body)
  out_t = run(tables, xt)
  return jnp.transpose(out_t, (1, 0, 2))


# speedup vs baseline: 3.1682x; 1.5587x over previous
import functools

import jax
import jax.numpy as jnp
from jax import lax
from jax.experimental import pallas as pl
from jax.experimental.pallas import tpu as pltpu
from jax.experimental.pallas import tpu_sc as plsc

_F = 26
_V1 = 100001
_D = 16
_B = 16384
_NC, _NS, _L = 2, 16, 16
_NW = _NC * _NS
_U = _F * _D
_UPW = _U // _NW


def _body(tab_hbm, x_hbm, out_hbm, xbuf, gbuf0, gbuf1, sem0, sem1):
  wid = lax.axis_index("s") * _NC + lax.axis_index("c")
  u0 = wid * _UPW
  bufs = (gbuf0, gbuf1)
  sems = (sem0, sem1)

  f_start = jnp.minimum(u0 // _D, _F - 2)
  pltpu.sync_copy(x_hbm.at[pl.ds(f_start, 2)], xbuf)

  def start(u, buf, sem):
    f = u // _D
    d = u - f * _D
    pltpu.async_copy(tab_hbm.at[f, d].at[xbuf.at[f - f_start]], buf, sem)

  start(u0, bufs[0], sems[0])
  for i in range(_UPW):
    u = u0 + i
    if i + 1 < _UPW:
      start(u + 1, bufs[(i + 1) % 2], sems[(i + 1) % 2])
    f = u // _D
    d = u - f * _D
    cur = i % 2
    pltpu.make_async_copy(
        tab_hbm.at[f, d].at[xbuf.at[f - f_start]], bufs[cur], sems[cur]
    ).wait()
    pltpu.sync_copy(bufs[cur], out_hbm.at[f, d])


def kernel(X, tables):
  tt = jnp.transpose(tables, (0, 2, 1))
  xt = jnp.transpose(X, (1, 0))
  mesh = plsc.VectorSubcoreMesh(core_axis_name="c", subcore_axis_name="s")
  run = functools.partial(
      pl.kernel,
      out_type=jax.ShapeDtypeStruct((_F, _D, _B), jnp.float32),
      mesh=mesh,
      compiler_params=pltpu.CompilerParams(use_tc_tiling_on_sc=False),
      scratch_types=[
          pltpu.VMEM((2, _B), jnp.int32),
          pltpu.VMEM((_B,), jnp.float32),
          pltpu.VMEM((_B,), jnp.float32),
          pltpu.SemaphoreType.DMA,
          pltpu.SemaphoreType.DMA,
      ],
  )(_body)
  out_t = run(tt, xt)
  return jnp.transpose(out_t, (2, 0, 1))
